# baseline (speedup 1.0000x reference)
import jax
import jax.numpy as jnp
from jax.experimental import pallas as pl


_TM = 256


def _interp_tile(s_ref, x_ref, o_ref):
    o_ref[...] = jnp.dot(s_ref[...], x_ref[...],
                         preferred_element_type=jnp.float32)


def kernel(x_coarse, interp_matrix):
    m, k = interp_matrix.shape
    n = x_coarse.shape[1]
    return pl.pallas_call(
        _interp_tile,
        grid=(m // _TM,),
        in_specs=[
            pl.BlockSpec((_TM, k), lambda i: (i, 0)),
            pl.BlockSpec((k, n), lambda i: (0, 0)),
        ],
        out_specs=pl.BlockSpec((_TM, n), lambda i: (i, 0)),
        out_shape=jax.ShapeDtypeStruct((m, n), jnp.float32),
    )(interp_matrix, x_coarse)


# speedup vs baseline: 1.2076x; 1.2076x over previous
import jax
import jax.numpy as jnp
from jax.experimental import pallas as pl
from jax.experimental.pallas import tpu as pltpu


_TM = 512


def _interp_tile(s_ref, x_ref, o_ref):
    o_ref[...] = jnp.dot(s_ref[...], x_ref[...],
                         preferred_element_type=jnp.float32)


def kernel(x_coarse, interp_matrix):
    m, k = interp_matrix.shape
    n = x_coarse.shape[1]
    return pl.pallas_call(
        _interp_tile,
        grid=(m // _TM,),
        in_specs=[
            pl.BlockSpec((_TM, k), lambda i: (i, 0)),
            pl.BlockSpec(memory_space=pltpu.MemorySpace.VMEM),
        ],
        out_specs=pl.BlockSpec((_TM, n), lambda i: (i, 0)),
        out_shape=jax.ShapeDtypeStruct((m, n), jnp.float32),
    )(interp_matrix, x_coarse)
